# baseline (speedup 1.0000x reference)
import functools

import jax
import jax.numpy as jnp
from jax import lax
from jax.experimental import pallas as pl
from jax.experimental.pallas import tpu as pltpu
from jax.experimental.pallas import tpu_sc as plsc

K = 26
NEMBED = 64
BATCH = 16384
NHID = 128
KN = K * NEMBED
NCG = KN // 128

NC, NS, L = 2, 16, 16
NW = NC * NS
CHUNK = 128
NSLAB = 2
SLAB = BATCH // NSLAB
BW = SLAB // NW
NCH = BW // CHUNK
T = K * NCH


def _sc_build(slab_start):
    mesh = plsc.VectorSubcoreMesh(
        core_axis_name="c", subcore_axis_name="s",
        num_cores=NC, num_subcores=NS)

    @functools.partial(
        pl.kernel,
        mesh=mesh,
        compiler_params=pltpu.CompilerParams(
            use_tc_tiling_on_sc=False, needs_layout_passes=False),
        out_type=jax.ShapeDtypeStruct((NCG, SLAB, 128), jnp.bfloat16),
        scratch_types=[
            pltpu.VMEM((2 * K, NCH, CHUNK), jnp.int32),
            pltpu.VMEM((CHUNK, NEMBED), jnp.float32),
            pltpu.VMEM((CHUNK, NEMBED), jnp.float32),
            pltpu.VMEM((CHUNK, NEMBED), jnp.float32),
            pltpu.VMEM((CHUNK, NEMBED), jnp.float32),
            pltpu.VMEM((CHUNK, NEMBED), jnp.bfloat16),
            pltpu.VMEM((CHUNK, NEMBED), jnp.bfloat16),
            pltpu.SemaphoreType.DMA,
            pltpu.SemaphoreType.DMA,
            pltpu.SemaphoreType.DMA,
            pltpu.SemaphoreType.DMA,
        ],
    )
    def sc_gather_mul(ctx_r, cnt_t, val_t, h_out, idx_all,
                      rc0, rv0, rc1, rv1, hb0, hb1, sg0, sg1, sw0, sw1):
        wid = lax.axis_index("s") * NC + lax.axis_index("c")
        crow = slab_start // CHUNK + wid * NCH

        pltpu.sync_copy(ctx_r.at[:, pl.ds(crow, NCH), :], idx_all)

        bufs = ((rc0, rv0, hb0, sg0, sw0), (rc1, rv1, hb1, sg1, sw1))

        def kj(t):
            return lax.shift_right_logical(t, 1), lax.bitwise_and(t, 1)

        def issue_gathers(t, rc, rv, sg):
            k, j = kj(t)
            pltpu.async_copy(cnt_t.at[idx_all.at[2 * k, j]], rc, sg)
            pltpu.async_copy(val_t.at[idx_all.at[2 * k + 1, j]], rv, sg)

        def wait_gathers(rc, rv, sg):
            pltpu.make_async_copy(cnt_t.at[idx_all.at[0, 0]], rc, sg).wait()
            pltpu.make_async_copy(val_t.at[idx_all.at[0, 0]], rv, sg).wait()

        def h_slice(t):
            k, j = kj(t)
            cg = lax.shift_right_logical(k, 1)
            half = lax.bitwise_and(k, 1) * NEMBED
            b0 = (wid * NCH + j) * CHUNK
            return h_out.at[cg, pl.ds(b0, CHUNK), pl.ds(half, NEMBED)]

        def wait_write(t, hb, sw):
            pltpu.make_async_copy(hb, h_slice(t), sw).wait()

        issue_gathers(0, rc0, rv0, sg0)

        @pl.loop(0, T, step=2)
        def _t0(t0):
            for b in range(2):
                rc, rv, hb, sg, sw = bufs[b]
                orc, orv, _, osg, _ = bufs[1 - b]
                t = t0 + b

                @pl.when(t + 1 < T)
                def _():
                    issue_gathers(t + 1, orc, orv, osg)

                wait_gathers(rc, rv, sg)

                @pl.when(t >= 2)
                def _():
                    wait_write(t - 2, hb, sw)

                @pl.loop(0, CHUNK, unroll=8)
                def _m(r):
                    for half in range(2):
                        pa = (rc[r, pl.ds(half * 32, L)]
                              * rv[r, pl.ds(half * 32, L)])
                        pb = (rc[r, pl.ds(half * 32 + L, L)]
                              * rv[r, pl.ds(half * 32 + L, L)])
                        hb[r, pl.ds(half * 32, 2 * L)] = plsc.pack(
                            pa, pb, format=plsc.PackFormat.INTERLEAVED)

                pltpu.async_copy(hb, h_slice(t), sw)

        wait_write(T - 2, bufs[(T - 2) % 2][2], bufs[(T - 2) % 2][4])
        wait_write(T - 1, bufs[(T - 1) % 2][2], bufs[(T - 1) % 2][4])

    return sc_gather_mul


_sc_slabs = tuple(_sc_build(s * SLAB) for s in range(NSLAB))


def _tc_body(h_ref, w_ref, b_ref, o_ref):
    acc = jnp.zeros(o_ref.shape, jnp.float32)
    for c in range(NCG):
        th = jnp.tanh(h_ref[c].astype(jnp.float32)).astype(jnp.bfloat16)
        acc += jax.lax.dot_general(
            th, w_ref[c], (((1,), (0,)), ((), ())),
            preferred_element_type=jnp.float32)
    o_ref[:] = acc + b_ref[:]


def _tc_mlp(h3, W3, b2):
    bB = 1024
    return pl.pallas_call(
        _tc_body,
        grid=(SLAB // bB,),
        in_specs=[
            pl.BlockSpec((NCG, bB, 128), lambda i: (0, i, 0)),
            pl.BlockSpec((NCG, 128, NHID), lambda i: (0, 0, 0)),
            pl.BlockSpec((1, NHID), lambda i: (0, 0)),
        ],
        out_specs=pl.BlockSpec((bB, NHID), lambda i: (i, 0)),
        out_shape=jax.ShapeDtypeStruct((SLAB, NHID), jnp.float32),
    )(h3, W3, b2)


def kernel(ctx, cnt_table, val_table, W, b):
    ctx_r = ctx.reshape(2 * K, BATCH // CHUNK, CHUNK)
    W3 = (W.reshape(K, 2, 2, 16, NHID)
          .transpose(0, 1, 3, 2, 4)
          .reshape(NCG, 128, NHID)
          .astype(jnp.bfloat16))
    b2 = b.reshape(1, NHID)
    hs = [_sc_slabs[s](ctx_r, cnt_table, val_table) for s in range(NSLAB)]
    outs = [_tc_mlp(h3, W3, b2) for h3 in hs]
    return jnp.concatenate(outs, axis=0)[None]


# speedup vs baseline: 1.8942x; 1.8942x over previous
import functools

import jax
import jax.numpy as jnp
from jax import lax
from jax.experimental import pallas as pl
from jax.experimental.pallas import tpu as pltpu
from jax.experimental.pallas import tpu_sc as plsc

K = 26
NEMBED = 64
BATCH = 16384
NHID = 128
KN = K * NEMBED
NCG = KN // 128

NC, NS, L = 2, 16, 16
NW = NC * NS
CHUNK = 128
NSLAB = 2
SLAB = BATCH // NSLAB
BW = SLAB // NW
NCH = BW // CHUNK
T = NCG * NCH


def _sc_build(slab_start):
    mesh = plsc.VectorSubcoreMesh(
        core_axis_name="c", subcore_axis_name="s",
        num_cores=NC, num_subcores=NS)

    @functools.partial(
        pl.kernel,
        mesh=mesh,
        compiler_params=pltpu.CompilerParams(
            use_tc_tiling_on_sc=False, needs_layout_passes=False),
        out_type=jax.ShapeDtypeStruct((NCG, SLAB // 2, 128), jnp.float32),
        scratch_types=[
            pltpu.VMEM((2 * K, NCH, CHUNK), jnp.int32),
            pltpu.VMEM((2, 4, CHUNK, NEMBED), jnp.float32),
            pltpu.VMEM((CHUNK // 2, 128), jnp.float32),
            pltpu.VMEM((CHUNK // 2, 128), jnp.float32),
            pltpu.SemaphoreType.DMA,
            pltpu.SemaphoreType.DMA,
            pltpu.SemaphoreType.DMA,
            pltpu.SemaphoreType.DMA,
        ],
    )
    def sc_gather_mul(ctx_r, cnt_t, val_t, h_out, idx_all, rows,
                      hb0, hb1, sg0, sg1, sw0, sw1):
        wid = lax.axis_index("s") * NC + lax.axis_index("c")
        crow = slab_start // CHUNK + wid * NCH

        pltpu.sync_copy(ctx_r.at[:, pl.ds(crow, NCH), :], idx_all)

        bufs = ((hb0, sg0, sw0), (hb1, sg1, sw1))

        def kj(t):
            return lax.shift_right_logical(t, 1), lax.bitwise_and(t, 1)

        def issue_gathers(t, rb, sg):
            cg, j = kj(t)
            for q, tab in ((0, cnt_t), (1, val_t), (2, cnt_t), (3, val_t)):
                pltpu.async_copy(tab.at[idx_all.at[4 * cg + q, j]],
                                 rows.at[rb, q], sg)

        def wait_gathers(rb, sg):
            for q in range(4):
                pltpu.make_async_copy(cnt_t.at[idx_all.at[0, 0]],
                                      rows.at[rb, q], sg).wait()

        def h_slice(t):
            cg, j = kj(t)
            r0 = (wid * NCH + j) * (CHUNK // 2)
            return h_out.at[cg, pl.ds(r0, CHUNK // 2), :]

        def wait_write(t, hb, sw):
            pltpu.make_async_copy(hb, h_slice(t), sw).wait()

        issue_gathers(0, 0, sg0)

        @pl.loop(0, T, step=2)
        def _t0(t0):
            for b in range(2):
                hb, sg, sw = bufs[b]
                _, osg, _ = bufs[1 - b]
                t = t0 + b

                @pl.when(t + 1 < T)
                def _():
                    issue_gathers(t + 1, 1 - b, osg)

                wait_gathers(b, sg)

                @pl.when(t >= 2)
                def _():
                    wait_write(t - 2, hb, sw)

                @pl.loop(0, CHUNK // 2, unroll=4)
                def _m(rr):
                    r = 2 * rr
                    for ke in range(2):
                        for seg in range(4):
                            s = pl.ds(seg * L, L)
                            pe = (rows[b, 2 * ke, r, s]
                                  * rows[b, 2 * ke + 1, r, s])
                            po = (rows[b, 2 * ke, r + 1, s]
                                  * rows[b, 2 * ke + 1, r + 1, s])
                            pk = plsc.pack(
                                pe, po, format=plsc.PackFormat.INTERLEAVED)
                            col = ke * 64 + seg * L
                            hb[rr, pl.ds(col, L)] = plsc.bitcast(
                                pk, jnp.float32)

                pltpu.async_copy(hb, h_slice(t), sw)

        wait_write(T - 2, bufs[(T - 2) % 2][0], bufs[(T - 2) % 2][2])
        wait_write(T - 1, bufs[(T - 1) % 2][0], bufs[(T - 1) % 2][2])

    return sc_gather_mul


_sc_slabs = tuple(_sc_build(s * SLAB) for s in range(NSLAB))


def _tc_body(h_ref, w_ref, b_ref, o_ref):
    bB = o_ref.shape[0]
    half = (bB // 2, NHID)
    acc_e = jnp.zeros(half, jnp.float32)
    acc_o = jnp.zeros(half, jnp.float32)
    for c in range(NCG):
        u = jax.lax.bitcast_convert_type(h_ref[c], jnp.uint32)
        xe = jax.lax.bitcast_convert_type(u << 16, jnp.float32)
        xo = jax.lax.bitcast_convert_type(
            u & jnp.uint32(0xFFFF0000), jnp.float32)
        te = jnp.tanh(xe).astype(jnp.bfloat16)
        to = jnp.tanh(xo).astype(jnp.bfloat16)
        dn = (((1,), (0,)), ((), ()))
        acc_e += jax.lax.dot_general(te, w_ref[c], dn,
                                     preferred_element_type=jnp.float32)
        acc_o += jax.lax.dot_general(to, w_ref[c], dn,
                                     preferred_element_type=jnp.float32)
    out = jnp.concatenate(
        [acc_e[:, None, :], acc_o[:, None, :]], axis=1).reshape(bB, NHID)
    o_ref[:] = out + b_ref[:]


def _tc_mlp(h3, W3, b2):
    bB = 1024
    return pl.pallas_call(
        _tc_body,
        grid=(SLAB // bB,),
        in_specs=[
            pl.BlockSpec((NCG, bB // 2, 128), lambda i: (0, i, 0)),
            pl.BlockSpec((NCG, 128, NHID), lambda i: (0, 0, 0)),
            pl.BlockSpec((1, NHID), lambda i: (0, 0)),
        ],
        out_specs=pl.BlockSpec((bB, NHID), lambda i: (i, 0)),
        out_shape=jax.ShapeDtypeStruct((SLAB, NHID), jnp.float32),
    )(h3, W3, b2)


def kernel(ctx, cnt_table, val_table, W, b):
    ctx_r = ctx.reshape(2 * K, BATCH // CHUNK, CHUNK)
    W3 = W.reshape(NCG, 128, NHID).astype(jnp.bfloat16)
    b2 = b.reshape(1, NHID)
    hs = [_sc_slabs[s](ctx_r, cnt_table, val_table) for s in range(NSLAB)]
    outs = [_tc_mlp(h3, W3, b2) for h3 in hs]
    return jnp.concatenate(outs, axis=0)[None]
